# Initial kernel scaffold; baseline (speedup 1.0000x reference)
import jax
import jax.numpy as jnp
from jax.experimental import pallas as pl


def kernel(x, edge_index, W_l0, W_r0, b0, W_l1, W_r1, b1, W_lin, b_lin):
    raise NotImplementedError("write your pallas kernel here")


# speedup vs baseline: 4.9046x; 4.9046x over previous
import functools

import jax
import jax.numpy as jnp
from jax import lax
from jax.experimental import pallas as pl
from jax.experimental.pallas import tpu as pltpu
from jax.experimental.pallas import tpu_sc as plsc

N = 10000
E = 320000
D = 128
NC = 2
NS = 16
NW = NC * NS
EPW = E // NW
CH = 80
NFULL = EPW // CH
NZCH = N // CH
CW = 16

def _sc_helpers(idx_z, rows):
  iota16 = lax.iota(jnp.int32, 16)

  def fill_rows(val16):
    def fr(i, _):
      def fc(j, _):
        rows[i, pl.ds(j * 16, 16)] = val16
        return 0
      lax.fori_loop(0, D // 16, fc, 0)
      return 0
    lax.fori_loop(0, CH, fr, 0)

  def fill_idx_z(row0):
    for v in range(CH // 16):
      idx_z[pl.ds(v * 16, 16)] = row0 + v * 16 + iota16

  return fill_rows, fill_idx_z


def _zero_table(tid, fill_idx_z, rows, acc, idx_z):
  def zchunk(j, _):
    c = tid + j * NS

    @pl.when(c < NZCH)
    def _():
      fill_idx_z(c * CH)
      pltpu.sync_copy(rows, acc.at[idx_z])
    return 0
  lax.fori_loop(0, (NZCH + NS - 1) // NS, zchunk, 0)


def _write_table(cid, tid, fill_idx_z, rows, acc, idx_z, outp, sem):
  def wchunk(j, _):
    c = tid + j * NS

    @pl.when(c < NZCH)
    def _():
      fill_idx_z(c * CH)
      pltpu.async_copy(acc.at[idx_z], rows, sem).wait()
      pltpu.sync_copy(rows, outp.at[cid, pl.ds(c * CH, CH)])
    return 0
  lax.fori_loop(0, (NZCH + NS - 1) // NS, wchunk, 0)


_MESH = plsc.VectorSubcoreMesh(core_axis_name="c", subcore_axis_name="s")

_EP_SCRATCH = (
    pltpu.VMEM((CH,), jnp.int32),
    pltpu.VMEM((CH,), jnp.int32),
    pltpu.VMEM((CH,), jnp.int32),
    pltpu.VMEM((CH, D), jnp.float32),
    pltpu.VMEM_SHARED((N, D), jnp.float32),
    pltpu.SemaphoreType.DMA,
)


def _edge_body(h, ei, outp, idx_s, idx_d, idx_z, rows, acc, sem):
  cid = lax.axis_index("c")
  tid = lax.axis_index("s")
  fill_rows, fill_idx_z = _sc_helpers(idx_z, rows)

  fill_rows(jnp.zeros((16,), jnp.float32))
  _zero_table(tid, fill_idx_z, rows, acc, idx_z)
  plsc.subcore_barrier()

  ebase = (cid * NS + tid) * EPW

  def chunk(c, _):
    base = pl.multiple_of(ebase + c * CH, 8)
    pltpu.sync_copy(ei.at[pl.ds(base, CH)], idx_s)
    pltpu.sync_copy(ei.at[pl.ds(E + base, CH)], idx_d)
    pltpu.async_copy(h.at[idx_s], rows, sem).wait()
    pltpu.sync_copy(rows, acc.at[idx_d], add=True)
    return 0
  lax.fori_loop(0, NFULL, chunk, 0)
  plsc.subcore_barrier()

  _write_table(cid, tid, fill_idx_z, rows, acc, idx_z, outp, sem)


def _count_body(ei, outc, idx_s, idx_d, idx_z, rows, acc, sem):
  cid = lax.axis_index("c")
  tid = lax.axis_index("s")
  fill_rows, fill_idx_z = _sc_helpers(idx_z, rows)

  fill_rows(jnp.zeros((16,), jnp.float32))
  _zero_table(tid, fill_idx_z, rows, acc, idx_z)
  plsc.subcore_barrier()

  fill_rows(jnp.ones((16,), jnp.float32))
  ebase = (cid * NS + tid) * EPW

  def chunk(c, _):
    base = pl.multiple_of(ebase + c * CH, 8)
    pltpu.sync_copy(ei.at[pl.ds(E + base, CH)], idx_d)
    pltpu.sync_copy(rows, acc.at[idx_d], add=True)
    return 0
  lax.fori_loop(0, NFULL, chunk, 0)
  plsc.subcore_barrier()

  _write_table(cid, tid, fill_idx_z, rows, acc, idx_z, outc, sem)


_edge_pass = pl.kernel(
    _edge_body,
    out_type=(jax.ShapeDtypeStruct((NC, N, D), jnp.float32),),
    mesh=_MESH, scratch_types=_EP_SCRATCH)

_count_pass = pl.kernel(
    _count_body,
    out_type=(jax.ShapeDtypeStruct((NC, N, D), jnp.float32),),
    mesh=_MESH, scratch_types=_EP_SCRATCH)

RB = 1000


def _mm0_body(p_ref, c_ref, x_ref, wl_ref, wr_ref, b_ref, o_ref):
  p = p_ref[...]
  c = c_ref[...]
  cnt = c[0, :, :1] + c[1, :, :1]
  agg = (p[0] + p[1]) / jnp.maximum(cnt, 1.0)
  o_ref[...] = (
      jnp.dot(agg, wl_ref[...], preferred_element_type=jnp.float32)
      + jnp.dot(x_ref[...], wr_ref[...], preferred_element_type=jnp.float32)
      + b_ref[...])


def _mm1_body(p_ref, c_ref, h1_ref, wl_ref, wr_ref, b_ref, wa_ref, wb_ref,
              bl_ref, o_ref):
  p = p_ref[...]
  c = c_ref[...]
  cnt = c[0, :, :1] + c[1, :, :1]
  agg = (p[0] + p[1]) / jnp.maximum(cnt, 1.0)
  h1 = h1_ref[...]
  h2 = (jnp.dot(agg, wl_ref[...], preferred_element_type=jnp.float32)
        + jnp.dot(h1, wr_ref[...], preferred_element_type=jnp.float32)
        + b_ref[...])
  o_ref[...] = (
      jnp.dot(h1, wa_ref[...], preferred_element_type=jnp.float32)
      + jnp.dot(h2, wb_ref[...], preferred_element_type=jnp.float32)
      + bl_ref[...])


def _full(shape):
  return pl.BlockSpec(shape, lambda i: tuple(0 for _ in shape))


_mm0 = pl.pallas_call(
    _mm0_body,
    grid=(N // RB,),
    in_specs=[
        pl.BlockSpec((NC, RB, D), lambda i: (0, i, 0)),
        pl.BlockSpec((NC, RB, D), lambda i: (0, i, 0)),
        pl.BlockSpec((RB, D), lambda i: (i, 0)),
        _full((D, D)),
        _full((D, D)),
        _full((1, D)),
    ],
    out_specs=pl.BlockSpec((RB, D), lambda i: (i, 0)),
    out_shape=jax.ShapeDtypeStruct((N, D), jnp.float32),
)

_mm1 = pl.pallas_call(
    _mm1_body,
    grid=(N // RB,),
    in_specs=[
        pl.BlockSpec((NC, RB, D), lambda i: (0, i, 0)),
        pl.BlockSpec((NC, RB, D), lambda i: (0, i, 0)),
        pl.BlockSpec((RB, D), lambda i: (i, 0)),
        _full((D, D)),
        _full((D, D)),
        _full((1, D)),
        _full((D, D)),
        _full((D, D)),
        _full((1, D)),
    ],
    out_specs=pl.BlockSpec((RB, D), lambda i: (i, 0)),
    out_shape=jax.ShapeDtypeStruct((N, D), jnp.float32),
)


@jax.jit
def kernel(x, edge_index, W_l0, W_r0, b0, W_l1, W_r1, b1, W_lin, b_lin):
  ei_flat = edge_index.reshape(2 * E)
  (cnt,) = _count_pass(ei_flat)
  (p0,) = _edge_pass(x, ei_flat)
  h1 = _mm0(p0, cnt, x, W_l0, W_r0, b0.reshape(1, D))
  (p1,) = _edge_pass(h1, ei_flat)
  out = _mm1(p1, cnt, h1, W_l1, W_r1, b1.reshape(1, D),
             W_lin[:D], W_lin[D:], b_lin.reshape(1, D))
  return out


# speedup vs baseline: 7.9641x; 1.6238x over previous
import functools

import jax
import jax.numpy as jnp
from jax import lax
from jax.experimental import pallas as pl
from jax.experimental.pallas import tpu as pltpu
from jax.experimental.pallas import tpu_sc as plsc

N = 10000
E = 320000
D = 128
NC = 2
NS = 16
NW = NC * NS
EPW = E // NW
CH = 80
NFULL = EPW // CH
NZCH = N // CH
CW = 16

def _sc_helpers(idx_z, rows):
  iota16 = lax.iota(jnp.int32, 16)

  def fill_rows(val16):
    def fr(i, _):
      def fc(j, _):
        rows[i, pl.ds(j * 16, 16)] = val16
        return 0
      lax.fori_loop(0, D // 16, fc, 0)
      return 0
    lax.fori_loop(0, CH, fr, 0)

  def fill_idx_z(row0):
    for v in range(CH // 16):
      idx_z[pl.ds(v * 16, 16)] = row0 + v * 16 + iota16

  return fill_rows, fill_idx_z


def _zero_table(tid, fill_idx_z, rows, acc, idx_z):
  def zchunk(j, _):
    c = tid + j * NS

    @pl.when(c < NZCH)
    def _():
      fill_idx_z(c * CH)
      pltpu.sync_copy(rows, acc.at[idx_z])
    return 0
  lax.fori_loop(0, (NZCH + NS - 1) // NS, zchunk, 0)


def _write_table(cid, tid, fill_idx_z, rows, acc, idx_z, outp, sem):
  def wchunk(j, _):
    c = tid + j * NS

    @pl.when(c < NZCH)
    def _():
      fill_idx_z(c * CH)
      pltpu.async_copy(acc.at[idx_z], rows, sem).wait()
      pltpu.sync_copy(rows, outp.at[cid, pl.ds(c * CH, CH)])
    return 0
  lax.fori_loop(0, (NZCH + NS - 1) // NS, wchunk, 0)


_MESH = plsc.VectorSubcoreMesh(core_axis_name="c", subcore_axis_name="s")

_EP_SCRATCH = (
    pltpu.VMEM((CH,), jnp.int32),
    pltpu.VMEM((CH,), jnp.int32),
    pltpu.VMEM((CH,), jnp.int32),
    pltpu.VMEM((CH,), jnp.int32),
    pltpu.VMEM((CH,), jnp.int32),
    pltpu.VMEM((CH, D), jnp.float32),
    pltpu.VMEM((CH, D), jnp.float32),
    pltpu.VMEM_SHARED((N, D), jnp.float32),
    pltpu.SemaphoreType.DMA,
    pltpu.SemaphoreType.DMA,
)


def _edge_body(h, ei, outp, idx_s0, idx_d0, idx_s1, idx_d1, idx_z,
               rows0, rows1, acc, sem0, sem1):
  cid = lax.axis_index("c")
  tid = lax.axis_index("s")
  fill_rows, fill_idx_z = _sc_helpers(idx_z, rows0)

  fill_rows(jnp.zeros((16,), jnp.float32))
  _zero_table(tid, fill_idx_z, rows0, acc, idx_z)
  plsc.subcore_barrier()

  ebase = (cid * NS + tid) * EPW
  bufs = ((idx_s0, idx_d0, rows0, sem0), (idx_s1, idx_d1, rows1, sem1))

  def issue(c, b):
    idx_s, idx_d, rows, sem = bufs[b]
    base = pl.multiple_of(ebase + c * CH, 8)
    pltpu.sync_copy(ei.at[pl.ds(base, CH)], idx_s)
    pltpu.sync_copy(ei.at[pl.ds(E + base, CH)], idx_d)
    pltpu.async_copy(h.at[idx_s], rows, sem)

  def complete(b):
    idx_s, idx_d, rows, sem = bufs[b]
    pltpu.make_async_copy(h.at[idx_s], rows, sem).wait()
    pltpu.sync_copy(rows, acc.at[idx_d], add=True)

  issue(0, 0)
  issue(1, 1)

  def pair(j, _):
    c = 2 * j
    complete(0)

    @pl.when(c + 2 < NFULL)
    def _():
      issue(c + 2, 0)
    complete(1)

    @pl.when(c + 3 < NFULL)
    def _():
      issue(c + 3, 1)
    return 0
  lax.fori_loop(0, NFULL // 2, pair, 0)
  if NFULL % 2:
    complete(0)
  plsc.subcore_barrier()

  _write_table(cid, tid, fill_idx_z, rows0, acc, idx_z, outp, sem0)


def _count_body(ei, outc, idx_s0, idx_d0, idx_s1, idx_d1, idx_z,
                rows0, rows1, acc, sem0, sem1):
  cid = lax.axis_index("c")
  tid = lax.axis_index("s")
  fill_rows, fill_idx_z = _sc_helpers(idx_z, rows0)

  fill_rows(jnp.zeros((16,), jnp.float32))
  _zero_table(tid, fill_idx_z, rows0, acc, idx_z)
  plsc.subcore_barrier()

  fill_rows(jnp.ones((16,), jnp.float32))
  ebase = (cid * NS + tid) * EPW
  bufs = ((idx_d0, sem0), (idx_d1, sem1))

  def issue(c, b):
    idx_d, sem = bufs[b]
    base = pl.multiple_of(ebase + c * CH, 8)
    pltpu.async_copy(ei.at[pl.ds(E + base, CH)], idx_d, sem)

  def complete(b):
    idx_d, sem = bufs[b]
    pltpu.make_async_copy(ei.at[pl.ds(0, CH)], idx_d, sem).wait()
    pltpu.sync_copy(rows0, acc.at[idx_d], add=True)

  issue(0, 0)
  issue(1, 1)

  def pair(j, _):
    c = 2 * j
    complete(0)

    @pl.when(c + 2 < NFULL)
    def _():
      issue(c + 2, 0)
    complete(1)

    @pl.when(c + 3 < NFULL)
    def _():
      issue(c + 3, 1)
    return 0
  lax.fori_loop(0, NFULL // 2, pair, 0)
  if NFULL % 2:
    complete(0)
  plsc.subcore_barrier()

  _write_table(cid, tid, fill_idx_z, rows0, acc, idx_z, outc, sem0)


_edge_pass = pl.kernel(
    _edge_body,
    out_type=(jax.ShapeDtypeStruct((NC, N, D), jnp.float32),),
    mesh=_MESH, scratch_types=_EP_SCRATCH)

_count_pass = pl.kernel(
    _count_body,
    out_type=(jax.ShapeDtypeStruct((NC, N, D), jnp.float32),),
    mesh=_MESH, scratch_types=_EP_SCRATCH)

RB = 1000


def _mm0_body(p_ref, c_ref, x_ref, wl_ref, wr_ref, b_ref, o_ref):
  p = p_ref[...]
  c = c_ref[...]
  cnt = c[0, :, :1] + c[1, :, :1]
  agg = (p[0] + p[1]) / jnp.maximum(cnt, 1.0)
  o_ref[...] = (
      jnp.dot(agg, wl_ref[...], preferred_element_type=jnp.float32)
      + jnp.dot(x_ref[...], wr_ref[...], preferred_element_type=jnp.float32)
      + b_ref[...])


def _mm1_body(p_ref, c_ref, h1_ref, wl_ref, wr_ref, b_ref, wa_ref, wb_ref,
              bl_ref, o_ref):
  p = p_ref[...]
  c = c_ref[...]
  cnt = c[0, :, :1] + c[1, :, :1]
  agg = (p[0] + p[1]) / jnp.maximum(cnt, 1.0)
  h1 = h1_ref[...]
  h2 = (jnp.dot(agg, wl_ref[...], preferred_element_type=jnp.float32)
        + jnp.dot(h1, wr_ref[...], preferred_element_type=jnp.float32)
        + b_ref[...])
  o_ref[...] = (
      jnp.dot(h1, wa_ref[...], preferred_element_type=jnp.float32)
      + jnp.dot(h2, wb_ref[...], preferred_element_type=jnp.float32)
      + bl_ref[...])


def _full(shape):
  return pl.BlockSpec(shape, lambda i: tuple(0 for _ in shape))


_mm0 = pl.pallas_call(
    _mm0_body,
    grid=(N // RB,),
    in_specs=[
        pl.BlockSpec((NC, RB, D), lambda i: (0, i, 0)),
        pl.BlockSpec((NC, RB, D), lambda i: (0, i, 0)),
        pl.BlockSpec((RB, D), lambda i: (i, 0)),
        _full((D, D)),
        _full((D, D)),
        _full((1, D)),
    ],
    out_specs=pl.BlockSpec((RB, D), lambda i: (i, 0)),
    out_shape=jax.ShapeDtypeStruct((N, D), jnp.float32),
)

_mm1 = pl.pallas_call(
    _mm1_body,
    grid=(N // RB,),
    in_specs=[
        pl.BlockSpec((NC, RB, D), lambda i: (0, i, 0)),
        pl.BlockSpec((NC, RB, D), lambda i: (0, i, 0)),
        pl.BlockSpec((RB, D), lambda i: (i, 0)),
        _full((D, D)),
        _full((D, D)),
        _full((1, D)),
        _full((D, D)),
        _full((D, D)),
        _full((1, D)),
    ],
    out_specs=pl.BlockSpec((RB, D), lambda i: (i, 0)),
    out_shape=jax.ShapeDtypeStruct((N, D), jnp.float32),
)


@jax.jit
def kernel(x, edge_index, W_l0, W_r0, b0, W_l1, W_r1, b1, W_lin, b_lin):
  ei_flat = edge_index.reshape(2 * E)
  (cnt,) = _count_pass(ei_flat)
  (p0,) = _edge_pass(x, ei_flat)
  h1 = _mm0(p0, cnt, x, W_l0, W_r0, b0.reshape(1, D))
  (p1,) = _edge_pass(h1, ei_flat)
  out = _mm1(p1, cnt, h1, W_l1, W_r1, b1.reshape(1, D),
             W_lin[:D], W_lin[D:], b_lin.reshape(1, D))
  return out


# speedup vs baseline: 10.9304x; 1.3724x over previous
import functools

import jax
import jax.numpy as jnp
from jax import lax
from jax.experimental import pallas as pl
from jax.experimental.pallas import tpu as pltpu
from jax.experimental.pallas import tpu_sc as plsc

N = 10000
E = 320000
D = 128
NC = 2
NS = 16
NW = NC * NS
EPW = E // NW
CH = 80
NFULL = EPW // CH
NZCH = N // CH
CW = 16

def _sc_helpers(idx_z, rows):
  iota16 = lax.iota(jnp.int32, 16)

  def fill_rows(val16):
    def fr(i, _):
      def fc(j, _):
        rows[i, pl.ds(j * 16, 16)] = val16
        return 0
      lax.fori_loop(0, D // 16, fc, 0)
      return 0
    lax.fori_loop(0, CH, fr, 0)

  def fill_idx_z(row0):
    for v in range(CH // 16):
      idx_z[pl.ds(v * 16, 16)] = row0 + v * 16 + iota16

  return fill_rows, fill_idx_z


def _zero_table(tid, fill_idx_z, rows, acc, idx_z):
  def zchunk(j, _):
    c = tid + j * NS

    @pl.when(c < NZCH)
    def _():
      fill_idx_z(c * CH)
      pltpu.sync_copy(rows, acc.at[idx_z])
    return 0
  lax.fori_loop(0, (NZCH + NS - 1) // NS, zchunk, 0)


def _write_table(cid, tid, fill_idx_z, rows, acc, idx_z, outp, sem):
  def wchunk(j, _):
    c = tid + j * NS

    @pl.when(c < NZCH)
    def _():
      fill_idx_z(c * CH)
      pltpu.async_copy(acc.at[idx_z], rows, sem).wait()
      pltpu.sync_copy(rows, outp.at[cid, pl.ds(c * CH, CH)])
    return 0
  lax.fori_loop(0, (NZCH + NS - 1) // NS, wchunk, 0)


_MESH = plsc.VectorSubcoreMesh(core_axis_name="c", subcore_axis_name="s")

_EP_SCRATCH = (
    pltpu.VMEM((CH,), jnp.int32),
    pltpu.VMEM((CH,), jnp.int32),
    pltpu.VMEM((CH,), jnp.int32),
    pltpu.VMEM((CH,), jnp.int32),
    pltpu.VMEM((CH,), jnp.int32),
    pltpu.VMEM((CH,), jnp.int32),
    pltpu.VMEM((CH,), jnp.int32),
    pltpu.VMEM((CH, D), jnp.float32),
    pltpu.VMEM((CH, D), jnp.float32),
    pltpu.VMEM_SHARED((N, D), jnp.float32),
    pltpu.SemaphoreType.DMA,
    pltpu.SemaphoreType.DMA,
    pltpu.SemaphoreType.DMA,
    pltpu.SemaphoreType.DMA,
    pltpu.SemaphoreType.DMA,
    pltpu.SemaphoreType.DMA,
    pltpu.SemaphoreType.DMA,
    pltpu.SemaphoreType.DMA,
)

_UNROLL = 6
_NMAIN = (NFULL // _UNROLL) * _UNROLL


def _edge_body(h, ei, outp, ixs0, ixs1, ixs2, ixd0, ixd1, ixd2, idx_z,
               rows0, rows1, acc,
               semg0, semg1, sis0, sis1, sis2, sid0, sid1, sid2):
  cid = lax.axis_index("c")
  tid = lax.axis_index("s")
  fill_rows, fill_idx_z = _sc_helpers(idx_z, rows0)

  fill_rows(jnp.zeros((16,), jnp.float32))
  _zero_table(tid, fill_idx_z, rows0, acc, idx_z)
  plsc.subcore_barrier()

  ebase = (cid * NS + tid) * EPW
  rows = (rows0, rows1)
  semg = (semg0, semg1)
  sis = (sis0, sis1, sis2)
  sid = (sid0, sid1, sid2)
  ixs = (ixs0, ixs1, ixs2)
  ixd = (ixd0, ixd1, ixd2)

  def issue_idx(c, s):
    base = pl.multiple_of(ebase + c * CH, 8)
    pltpu.async_copy(ei.at[pl.ds(base, CH)], ixs[s], sis[s])
    pltpu.async_copy(ei.at[pl.ds(E + base, CH)], ixd[s], sid[s])

  def wait_is(s):
    pltpu.make_async_copy(ei.at[pl.ds(0, CH)], ixs[s], sis[s]).wait()

  def wait_id(s):
    pltpu.make_async_copy(ei.at[pl.ds(0, CH)], ixd[s], sid[s]).wait()

  def issue_gather(s, b):
    pltpu.async_copy(h.at[ixs[s]], rows[b], semg[b])

  for k in range(3):
    issue_idx(k, k)
  for k in range(2):
    wait_is(k)
    issue_gather(k, k)

  def six(j, _):
    c0 = _UNROLL * j
    for k in range(_UNROLL):
      b, s = k % 2, k % 3
      wait_id(s)
      pltpu.make_async_copy(h.at[ixs[s]], rows[b], semg[b]).wait()
      pltpu.sync_copy(rows[b], acc.at[ixd[s]], add=True)
      wait_is((k + 2) % 3)
      issue_gather((k + 2) % 3, b)
      issue_idx(c0 + k + 3, s)
    return 0
  lax.fori_loop(0, _NMAIN // _UNROLL, six, 0)

  for c in range(_NMAIN, NFULL):
    k = c - _NMAIN
    b, s = c % 2, c % 3
    wait_id(s)
    pltpu.make_async_copy(h.at[ixs[s]], rows[b], semg[b]).wait()
    pltpu.sync_copy(rows[b], acc.at[ixd[s]], add=True)
    if c + 2 < NFULL:
      wait_is((c + 2) % 3)
      issue_gather((c + 2) % 3, b)
    if c + 3 < NFULL:
      issue_idx(c + 3, s)
  plsc.subcore_barrier()

  _write_table(cid, tid, fill_idx_z, rows0, acc, idx_z, outp, semg0)


_CNT_SCRATCH = (
    pltpu.VMEM((CH,), jnp.int32),
    pltpu.VMEM((CH,), jnp.int32),
    pltpu.VMEM((CH,), jnp.int32),
    pltpu.VMEM((CH, D), jnp.float32),
    pltpu.VMEM_SHARED((N, D), jnp.float32),
    pltpu.SemaphoreType.DMA,
    pltpu.SemaphoreType.DMA,
)


def _count_body(ei, outc, idx_d0, idx_d1, idx_z, rows0, acc, sem0, sem1):
  cid = lax.axis_index("c")
  tid = lax.axis_index("s")
  fill_rows, fill_idx_z = _sc_helpers(idx_z, rows0)

  fill_rows(jnp.zeros((16,), jnp.float32))
  _zero_table(tid, fill_idx_z, rows0, acc, idx_z)
  plsc.subcore_barrier()

  fill_rows(jnp.ones((16,), jnp.float32))
  ebase = (cid * NS + tid) * EPW
  bufs = ((idx_d0, sem0), (idx_d1, sem1))

  def issue(c, b):
    idx_d, sem = bufs[b]
    base = pl.multiple_of(ebase + c * CH, 8)
    pltpu.async_copy(ei.at[pl.ds(E + base, CH)], idx_d, sem)

  def complete(b):
    idx_d, sem = bufs[b]
    pltpu.make_async_copy(ei.at[pl.ds(0, CH)], idx_d, sem).wait()
    pltpu.sync_copy(rows0, acc.at[idx_d], add=True)

  issue(0, 0)
  issue(1, 1)

  def pair(j, _):
    c = 2 * j
    complete(0)

    @pl.when(c + 2 < NFULL)
    def _():
      issue(c + 2, 0)
    complete(1)

    @pl.when(c + 3 < NFULL)
    def _():
      issue(c + 3, 1)
    return 0
  lax.fori_loop(0, NFULL // 2, pair, 0)
  if NFULL % 2:
    complete(0)
  plsc.subcore_barrier()

  _write_table(cid, tid, fill_idx_z, rows0, acc, idx_z, outc, sem0)


_edge_pass = pl.kernel(
    _edge_body,
    out_type=(jax.ShapeDtypeStruct((NC, N, D), jnp.float32),),
    mesh=_MESH, scratch_types=_EP_SCRATCH)

_count_pass = pl.kernel(
    _count_body,
    out_type=(jax.ShapeDtypeStruct((NC, N, D), jnp.float32),),
    mesh=_MESH, scratch_types=_CNT_SCRATCH)

RB = 1000


def _mm0_body(p_ref, c_ref, x_ref, wl_ref, wr_ref, b_ref, o_ref):
  p = p_ref[...]
  c = c_ref[...]
  cnt = c[0, :, :1] + c[1, :, :1]
  agg = (p[0] + p[1]) / jnp.maximum(cnt, 1.0)
  o_ref[...] = (
      jnp.dot(agg, wl_ref[...], preferred_element_type=jnp.float32)
      + jnp.dot(x_ref[...], wr_ref[...], preferred_element_type=jnp.float32)
      + b_ref[...])


def _mm1_body(p_ref, c_ref, h1_ref, wl_ref, wr_ref, b_ref, wa_ref, wb_ref,
              bl_ref, o_ref):
  p = p_ref[...]
  c = c_ref[...]
  cnt = c[0, :, :1] + c[1, :, :1]
  agg = (p[0] + p[1]) / jnp.maximum(cnt, 1.0)
  h1 = h1_ref[...]
  h2 = (jnp.dot(agg, wl_ref[...], preferred_element_type=jnp.float32)
        + jnp.dot(h1, wr_ref[...], preferred_element_type=jnp.float32)
        + b_ref[...])
  o_ref[...] = (
      jnp.dot(h1, wa_ref[...], preferred_element_type=jnp.float32)
      + jnp.dot(h2, wb_ref[...], preferred_element_type=jnp.float32)
      + bl_ref[...])


def _full(shape):
  return pl.BlockSpec(shape, lambda i: tuple(0 for _ in shape))


_mm0 = pl.pallas_call(
    _mm0_body,
    grid=(N // RB,),
    in_specs=[
        pl.BlockSpec((NC, RB, D), lambda i: (0, i, 0)),
        pl.BlockSpec((NC, RB, D), lambda i: (0, i, 0)),
        pl.BlockSpec((RB, D), lambda i: (i, 0)),
        _full((D, D)),
        _full((D, D)),
        _full((1, D)),
    ],
    out_specs=pl.BlockSpec((RB, D), lambda i: (i, 0)),
    out_shape=jax.ShapeDtypeStruct((N, D), jnp.float32),
)

_mm1 = pl.pallas_call(
    _mm1_body,
    grid=(N // RB,),
    in_specs=[
        pl.BlockSpec((NC, RB, D), lambda i: (0, i, 0)),
        pl.BlockSpec((NC, RB, D), lambda i: (0, i, 0)),
        pl.BlockSpec((RB, D), lambda i: (i, 0)),
        _full((D, D)),
        _full((D, D)),
        _full((1, D)),
        _full((D, D)),
        _full((D, D)),
        _full((1, D)),
    ],
    out_specs=pl.BlockSpec((RB, D), lambda i: (i, 0)),
    out_shape=jax.ShapeDtypeStruct((N, D), jnp.float32),
)


@jax.jit
def kernel(x, edge_index, W_l0, W_r0, b0, W_l1, W_r1, b1, W_lin, b_lin):
  ei_flat = edge_index.reshape(2 * E)
  (cnt,) = _count_pass(ei_flat)
  (p0,) = _edge_pass(x, ei_flat)
  h1 = _mm0(p0, cnt, x, W_l0, W_r0, b0.reshape(1, D))
  (p1,) = _edge_pass(h1, ei_flat)
  out = _mm1(p1, cnt, h1, W_l1, W_r1, b1.reshape(1, D),
             W_lin[:D], W_lin[D:], b_lin.reshape(1, D))
  return out
